# baseline (speedup 1.0000x reference)
import jax
import jax.numpy as jnp
from jax.experimental import pallas as pl

_W_T = 0.1
_W_M = 0.1
_RB = 8


def _score_block(q_ref, k_ref, tl_ref, ms_ref, out_ref):
    RB, C, T = k_ref.shape
    qn = q_ref[:, 0, :]
    kt = k_ref[...]

    p = kt * kt
    acc = p[:, 0:8, :]
    for g in range(1, 8):
        acc = acc + p[:, 8 * g:8 * g + 8, :]
    t4 = acc[:, 0:4, :] + acc[:, 4:8, :]
    t2 = t4[:, 0:2, :] + t4[:, 2:4, :]
    n2 = t2[:, 0:1, :] + t2[:, 1:2, :]
    n = jnp.sqrt(n2) + 1e-6
    kb = (kt / n).astype(jnp.bfloat16)

    qb = qn.astype(jnp.bfloat16)
    rows = []
    for r in range(RB):
        s_all = jax.lax.dot_general(qb, kb[r], (((1,), (0,)), ((), ())),
                                    preferred_element_type=jnp.float32)
        rows.append(s_all[r:r + 1, :])
    s = jnp.concatenate(rows, axis=0)

    s = s + _W_T * jnp.exp(-jnp.maximum(tl_ref[:, 0, :], 0.0))
    s = s + _W_M * ms_ref[:, 0, :]

    iota = jax.lax.broadcasted_iota(jnp.int32, (RB, T), 1)
    col = jax.lax.broadcasted_iota(jnp.int32, (RB, 64), 1)
    out_idx = jnp.zeros((RB, 64), jnp.int32)
    neg_inf = jnp.float32(float("-inf"))
    for j in range(64):
        m = jnp.max(s, axis=1, keepdims=True)
        cand = jnp.where(s == m, iota, jnp.int32(T))
        pos = jnp.min(cand, axis=1, keepdims=True)
        out_idx = jnp.where(col == j, pos, out_idx)
        s = jnp.where(iota == pos, neg_inf, s)
    out_ref[:, 0, :] = out_idx


def kernel(q_t, K_hist, top_r, top_u, time_lag, motion_score):
    B, P, T, C = K_hist.shape
    R = B * P
    qn = q_t / (jnp.linalg.norm(q_t, axis=-1, keepdims=True) + 1e-06)

    top_idx = pl.pallas_call(
        _score_block,
        grid=(R // _RB,),
        in_specs=[
            pl.BlockSpec((_RB, 1, C), lambda i: (i, 0, 0)),
            pl.BlockSpec((_RB, C, T), lambda i: (i, 0, 0)),
            pl.BlockSpec((_RB, 1, T), lambda i: (i, 0, 0)),
            pl.BlockSpec((_RB, 1, T), lambda i: (i, 0, 0)),
        ],
        out_specs=pl.BlockSpec((_RB, 1, 64), lambda i: (i, 0, 0)),
        out_shape=jax.ShapeDtypeStruct((R, 1, 64), jnp.int32),
    )(qn.reshape(R, 1, C),
      jnp.swapaxes(K_hist, 2, 3).reshape(R, C, T),
      time_lag.reshape(R, 1, T), motion_score.reshape(R, 1, T))

    read_idx = top_idx.reshape(R, 64)
    write_idx = read_idx[:, : min(16, T)]
    return (read_idx.reshape(B, P, -1), write_idx.reshape(B, P, -1))


# speedup vs baseline: 3.4292x; 3.4292x over previous
import jax
import jax.numpy as jnp
from jax.experimental import pallas as pl

_W_T = 0.1
_W_M = 0.1
_RB = 16


def _score_block(q_ref, k_ref, tl_ref, ms_ref, out_ref):
    RB, C, T = k_ref.shape
    qn = q_ref[:, 0, :]
    kt = k_ref[...]

    p = kt * kt
    acc = p[:, 0:8, :]
    for g in range(1, 8):
        acc = acc + p[:, 8 * g:8 * g + 8, :]
    t4 = acc[:, 0:4, :] + acc[:, 4:8, :]
    t2 = t4[:, 0:2, :] + t4[:, 2:4, :]
    n2 = t2[:, 0:1, :] + t2[:, 1:2, :]
    n = jnp.sqrt(n2) + 1e-6
    kb = (kt / n).astype(jnp.bfloat16)

    qb = qn.astype(jnp.bfloat16)
    rows = []
    for r in range(RB):
        s_all = jax.lax.dot_general(qb, kb[r], (((1,), (0,)), ((), ())),
                                    preferred_element_type=jnp.float32)
        rows.append(s_all[r:r + 1, :])
    s = jnp.concatenate(rows, axis=0)

    s = s + _W_T * jnp.exp(-jnp.maximum(tl_ref[:, 0, :], 0.0))
    s = s + _W_M * ms_ref[:, 0, :]
    out_ref[:, 0, :] = s


def kernel(q_t, K_hist, top_r, top_u, time_lag, motion_score):
    B, P, T, C = K_hist.shape
    R = B * P
    qn = q_t / (jnp.linalg.norm(q_t, axis=-1, keepdims=True) + 1e-06)

    scores = pl.pallas_call(
        _score_block,
        grid=(R // _RB,),
        in_specs=[
            pl.BlockSpec((_RB, 1, C), lambda i: (i, 0, 0)),
            pl.BlockSpec((_RB, C, T), lambda i: (i, 0, 0)),
            pl.BlockSpec((_RB, 1, T), lambda i: (i, 0, 0)),
            pl.BlockSpec((_RB, 1, T), lambda i: (i, 0, 0)),
        ],
        out_specs=pl.BlockSpec((_RB, 1, T), lambda i: (i, 0, 0)),
        out_shape=jax.ShapeDtypeStruct((R, 1, T), jnp.float32),
    )(qn.reshape(R, 1, C),
      jnp.swapaxes(K_hist, 2, 3).reshape(R, C, T),
      time_lag.reshape(R, 1, T), motion_score.reshape(R, 1, T))

    read_idx = jax.lax.top_k(scores.reshape(R, T), min(64, T))[1]
    write_idx = read_idx[:, : min(16, T)]
    return (read_idx.reshape(B, P, -1), write_idx.reshape(B, P, -1))


# speedup vs baseline: 3.4649x; 1.0104x over previous
import jax
import jax.numpy as jnp
from jax.experimental import pallas as pl

_W_T = 0.1
_W_M = 0.1
_RB = 32


def _score_block(q_ref, k_ref, tl_ref, ms_ref, out_ref):
    RB, C, T = k_ref.shape
    qn = q_ref[:, 0, :]
    kt = k_ref[...]

    p = kt * kt
    acc = p[:, 0:8, :]
    for g in range(1, 8):
        acc = acc + p[:, 8 * g:8 * g + 8, :]
    t4 = acc[:, 0:4, :] + acc[:, 4:8, :]
    t2 = t4[:, 0:2, :] + t4[:, 2:4, :]
    n2 = t2[:, 0:1, :] + t2[:, 1:2, :]
    n = jnp.sqrt(n2) + 1e-6
    kb = (kt / n).astype(jnp.bfloat16)

    qb = qn.astype(jnp.bfloat16)
    rows = []
    for r in range(RB):
        s_all = jax.lax.dot_general(qb, kb[r], (((1,), (0,)), ((), ())),
                                    preferred_element_type=jnp.float32)
        rows.append(s_all[r:r + 1, :])
    s = jnp.concatenate(rows, axis=0)

    s = s + _W_T * jnp.exp(-jnp.maximum(tl_ref[:, 0, :], 0.0))
    s = s + _W_M * ms_ref[:, 0, :]
    out_ref[:, 0, :] = s


def kernel(q_t, K_hist, top_r, top_u, time_lag, motion_score):
    B, P, T, C = K_hist.shape
    R = B * P
    qn = q_t / (jnp.linalg.norm(q_t, axis=-1, keepdims=True) + 1e-06)

    scores = pl.pallas_call(
        _score_block,
        grid=(R // _RB,),
        in_specs=[
            pl.BlockSpec((_RB, 1, C), lambda i: (i, 0, 0)),
            pl.BlockSpec((_RB, C, T), lambda i: (i, 0, 0)),
            pl.BlockSpec((_RB, 1, T), lambda i: (i, 0, 0)),
            pl.BlockSpec((_RB, 1, T), lambda i: (i, 0, 0)),
        ],
        out_specs=pl.BlockSpec((_RB, 1, T), lambda i: (i, 0, 0)),
        out_shape=jax.ShapeDtypeStruct((R, 1, T), jnp.float32),
    )(qn.reshape(R, 1, C),
      jnp.swapaxes(K_hist, 2, 3).reshape(R, C, T),
      time_lag.reshape(R, 1, T), motion_score.reshape(R, 1, T))

    read_idx = jax.lax.top_k(scores.reshape(R, T), min(64, T))[1]
    write_idx = read_idx[:, : min(16, T)]
    return (read_idx.reshape(B, P, -1), write_idx.reshape(B, P, -1))
